# baseline (speedup 1.0000x reference)
import jax
import jax.numpy as jnp
from jax import lax
from jax.experimental import pallas as pl
from jax.experimental.pallas import tpu as pltpu
from jax.experimental.pallas import tpu_sc as plsc


_BS = 2048


def _add_kernel(x_ref, pe_ref, o_ref):
    o_ref[0] = x_ref[0] + pe_ref[...]


def _tc_kernel(x, amp_table):
    B, S, D = x.shape
    grid = (S // _BS, B)
    return pl.pallas_call(
        _add_kernel,
        grid=grid,
        in_specs=[
            pl.BlockSpec((1, _BS, D), lambda i, j: (j, i, 0)),
            pl.BlockSpec((_BS, D), lambda i, j: (i, 0)),
        ],
        out_specs=pl.BlockSpec((1, _BS, D), lambda i, j: (j, i, 0)),
        out_shape=jax.ShapeDtypeStruct((B, S, D), x.dtype),
        compiler_params=pltpu.CompilerParams(
            dimension_semantics=("parallel", "parallel"),
        ),
    )(x, amp_table)


_NC, _NS = 2, 16
_NW = _NC * _NS
_CH = 16384


def _sc_body(total_words, pe_words, x_hbm, pe_hbm, out_hbm, x_v, pe_v):
    w = lax.axis_index("s") * _NC + lax.axis_index("c")
    words_per_w = total_words // _NW
    base = w * words_per_w
    pe_base = lax.rem(base, pe_words)
    n_chunks = words_per_w // _CH

    def chunk(c, carry):
        off = base + c * _CH
        pe_off = pe_base + c * _CH
        pltpu.sync_copy(x_hbm.at[pl.ds(off, _CH)], x_v)
        pltpu.sync_copy(pe_hbm.at[pl.ds(pe_off, _CH)], pe_v)

        def add16(k, carry2):
            sl = pl.ds(k * 16, 16)
            x_v[sl] = x_v[sl] + pe_v[sl]
            return carry2

        lax.fori_loop(0, _CH // 16, add16, 0)
        pltpu.sync_copy(x_v, out_hbm.at[pl.ds(off, _CH)])
        return carry

    lax.fori_loop(0, n_chunks, chunk, 0)


def _sc_kernel(x, amp_table):
    B, S, D = x.shape
    total_words = B * S * D
    pe_words = S * D
    xf = x.reshape(total_words)
    pef = amp_table.reshape(pe_words)
    import functools
    body = functools.partial(_sc_body, total_words, pe_words)
    kf = pl.kernel(
        body,
        out_type=jax.ShapeDtypeStruct((total_words,), jnp.float32),
        mesh=plsc.VectorSubcoreMesh(core_axis_name="c", subcore_axis_name="s"),
        scratch_types=[
            pltpu.VMEM((_CH,), jnp.float32),
            pltpu.VMEM((_CH,), jnp.float32),
        ],
    )
    return kf(xf, pef).reshape(B, S, D)


def kernel(x, amp_table):
    return _sc_kernel(x, amp_table)


# speedup vs baseline: 1.3010x; 1.3010x over previous
import jax
import jax.numpy as jnp
from jax import lax
from jax.experimental import pallas as pl
from jax.experimental.pallas import tpu as pltpu
from jax.experimental.pallas import tpu_sc as plsc


_BS = 2048


def _add_kernel(x_ref, pe_ref, o_ref):
    o_ref[0] = x_ref[0] + pe_ref[...]


def _tc_kernel(x, amp_table):
    B, S, D = x.shape
    grid = (S // _BS, B)
    return pl.pallas_call(
        _add_kernel,
        grid=grid,
        in_specs=[
            pl.BlockSpec((1, _BS, D), lambda i, j: (j, i, 0)),
            pl.BlockSpec((_BS, D), lambda i, j: (i, 0)),
        ],
        out_specs=pl.BlockSpec((1, _BS, D), lambda i, j: (j, i, 0)),
        out_shape=jax.ShapeDtypeStruct((B, S, D), x.dtype),
        compiler_params=pltpu.CompilerParams(
            dimension_semantics=("parallel", "parallel"),
        ),
    )(x, amp_table)


_NC, _NS = 2, 16
_NW = _NC * _NS
_CHD = 8192
_NBUF = 4
_UNROLL = 4


def _sc_body(total_words, pe_words, x_hbm, pe_hbm, out_hbm, x_v, pe_v, rsx, rsp, ws):
    w = lax.axis_index("s") * _NC + lax.axis_index("c")
    words_per_w = total_words // _NW
    base = w * words_per_w
    pe_base = lax.rem(base, pe_words)
    n = words_per_w // _CHD

    def start_read(c, b):
        off = base + c * _CHD
        pltpu.async_copy(x_hbm.at[pl.ds(off, _CHD)], x_v.at[b], rsx.at[b])
        pltpu.async_copy(
            pe_hbm.at[pl.ds(pe_base + c * _CHD, _CHD)], pe_v.at[b], rsp.at[b]
        )

    for b in range(_NBUF - 1):
        start_read(b, b)

    def group(g, carry):
        for u in range(_NBUF):
            c = g * _NBUF + u
            off = base + c * _CHD
            pltpu.make_async_copy(
                x_hbm.at[pl.ds(off, _CHD)], x_v.at[u], rsx.at[u]
            ).wait()
            pltpu.make_async_copy(
                pe_hbm.at[pl.ds(pe_base + c * _CHD, _CHD)], pe_v.at[u], rsp.at[u]
            ).wait()

            def add16(k, carry2, _u=u):
                for j in range(_UNROLL):
                    sl = pl.ds(k * (16 * _UNROLL) + j * 16, 16)
                    x_v[_u, sl] = x_v[_u, sl] + pe_v[_u, sl]
                return carry2

            lax.fori_loop(0, _CHD // (16 * _UNROLL), add16, 0)
            pltpu.async_copy(x_v.at[u], out_hbm.at[pl.ds(off, _CHD)], ws.at[u])

            b2 = (u + _NBUF - 1) % _NBUF
            off_prev = base + (c - 1) * _CHD

            @pl.when(c >= 1)
            def _():
                pltpu.make_async_copy(
                    x_v.at[b2], out_hbm.at[pl.ds(off_prev, _CHD)], ws.at[b2]
                ).wait()

            @pl.when(c + (_NBUF - 1) < n)
            def _():
                start_read(c + (_NBUF - 1), b2)
        return carry

    lax.fori_loop(0, n // _NBUF, group, 0)
    bl = (n - 1) % _NBUF
    pltpu.make_async_copy(
        x_v.at[bl], out_hbm.at[pl.ds(base + (n - 1) * _CHD, _CHD)], ws.at[bl]
    ).wait()


def _sc_kernel(x, amp_table):
    B, S, D = x.shape
    total_words = B * S * D
    pe_words = S * D
    xf = x.reshape(total_words)
    pef = amp_table.reshape(pe_words)
    import functools
    body = functools.partial(_sc_body, total_words, pe_words)
    kf = pl.kernel(
        body,
        out_type=jax.ShapeDtypeStruct((total_words,), jnp.float32),
        mesh=plsc.VectorSubcoreMesh(core_axis_name="c", subcore_axis_name="s"),
        scratch_types=[
            pltpu.VMEM((_NBUF, _CHD), jnp.float32),
            pltpu.VMEM((_NBUF, _CHD), jnp.float32),
            pltpu.SemaphoreType.DMA((_NBUF,)),
            pltpu.SemaphoreType.DMA((_NBUF,)),
            pltpu.SemaphoreType.DMA((_NBUF,)),
        ],
    )
    return kf(xf, pef).reshape(B, S, D)


def kernel(x, amp_table):
    return _sc_kernel(x, amp_table)


# speedup vs baseline: 7.7197x; 5.9339x over previous
import jax
import jax.numpy as jnp
from jax.experimental import pallas as pl
from jax.experimental.pallas import tpu as pltpu

_BS = 512


def _add_kernel(x_ref, pe_ref, o_ref):
    i = pl.program_id(0)
    o_ref[0] = x_ref[0] + pe_ref[pl.ds(i * _BS, _BS), :]


def kernel(x, amp_table):
    B, S, D = x.shape
    grid = (S // _BS, B)
    return pl.pallas_call(
        _add_kernel,
        grid=grid,
        in_specs=[
            pl.BlockSpec((1, _BS, D), lambda i, j: (j, i, 0)),
            pl.BlockSpec((S, D), lambda i, j: (0, 0)),
        ],
        out_specs=pl.BlockSpec((1, _BS, D), lambda i, j: (j, i, 0)),
        out_shape=jax.ShapeDtypeStruct((B, S, D), x.dtype),
        compiler_params=pltpu.CompilerParams(
            dimension_semantics=("arbitrary", "arbitrary"),
        ),
    )(x, amp_table)


# speedup vs baseline: 8.5626x; 1.1092x over previous
import jax
import jax.numpy as jnp
from jax.experimental import pallas as pl

_BS = 2048


def _add_kernel(x_ref, pe_ref, o_ref):
    o_ref[0] = x_ref[0] + pe_ref[...]


def kernel(x, amp_table):
    B, S, D = x.shape
    grid = (S // _BS, B)
    return pl.pallas_call(
        _add_kernel,
        grid=grid,
        in_specs=[
            pl.BlockSpec((1, _BS, D), lambda i, j: (j, i, 0)),
            pl.BlockSpec((_BS, D), lambda i, j: (i, 0)),
        ],
        out_specs=pl.BlockSpec((1, _BS, D), lambda i, j: (j, i, 0)),
        out_shape=jax.ShapeDtypeStruct((B, S, D), x.dtype),
    )(x, amp_table)


# speedup vs baseline: 8.5648x; 1.0003x over previous
import jax
import jax.numpy as jnp
from jax.experimental import pallas as pl
from jax.experimental.pallas import tpu as pltpu

_BS = 2048


def _add_kernel(x_ref, pe_hbm, o_ref, pb, sem):
    i = pl.program_id(0)
    j = pl.program_id(1)
    ni = pl.num_programs(0)

    @pl.when(jnp.logical_and(i == 0, j == 0))
    def _():
        pltpu.make_async_copy(
            pe_hbm.at[pl.ds(0, _BS)], pb.at[0], sem.at[0]
        ).start()

    @pl.when(jnp.logical_and(i + 1 < ni, j == 0))
    def _():
        pltpu.make_async_copy(
            pe_hbm.at[pl.ds((i + 1) * _BS, _BS)], pb.at[(i + 1) % 2], sem.at[(i + 1) % 2]
        ).start()

    @pl.when(j == 0)
    def _():
        pltpu.make_async_copy(
            pe_hbm.at[pl.ds(i * _BS, _BS)], pb.at[i % 2], sem.at[i % 2]
        ).wait()

    o_ref[0] = x_ref[0] + pb[i % 2]


def kernel(x, amp_table):
    B, S, D = x.shape
    grid = (S // _BS, B)
    return pl.pallas_call(
        _add_kernel,
        grid=grid,
        in_specs=[
            pl.BlockSpec((1, _BS, D), lambda i, j: (j, i, 0)),
            pl.BlockSpec(memory_space=pltpu.HBM),
        ],
        out_specs=pl.BlockSpec((1, _BS, D), lambda i, j: (j, i, 0)),
        out_shape=jax.ShapeDtypeStruct((B, S, D), x.dtype),
        scratch_shapes=[
            pltpu.VMEM((2, _BS, D), jnp.float32),
            pltpu.SemaphoreType.DMA((2,)),
        ],
    )(x, amp_table)


# speedup vs baseline: 8.5667x; 1.0002x over previous
import jax
import jax.numpy as jnp
from jax.experimental import pallas as pl

_BS = 2048


def _add_kernel(x_ref, pe_ref, o_ref):
    o_ref[0] = x_ref[0] + pe_ref[...]


def kernel(x, amp_table):
    B, S, D = x.shape
    grid = (S // _BS, B)
    return pl.pallas_call(
        _add_kernel,
        grid=grid,
        in_specs=[
            pl.BlockSpec((1, _BS, D), lambda i, j: (j, i, 0)),
            pl.BlockSpec((_BS, D), lambda i, j: (i, 0)),
        ],
        out_specs=pl.BlockSpec((1, _BS, D), lambda i, j: (j, i, 0)),
        out_shape=jax.ShapeDtypeStruct((B, S, D), x.dtype),
    )(x, amp_table)
